# baseline (speedup 1.0000x reference)
import functools
import math

import jax
import jax.numpy as jnp
from jax import lax
from jax.experimental import pallas as pl
from jax.experimental.pallas import tpu as pltpu
from jax.experimental.pallas import tpu_sc as plsc

HIDDEN = 64
SCALE = math.sqrt(HIDDEN)
LANES = 16
NCORES = 2
NSUB = 16
NW = NCORES * NSUB

CHUNK = 512


def _make_kernel(n_rows: int):
  assert n_rows % (NW * CHUNK) == 0
  rows_per_w = n_rows // NW
  nchunks = rows_per_w // CHUNK
  vpr = HIDDEN // LANES

  mesh = plsc.VectorSubcoreMesh(core_axis_name="c", subcore_axis_name="s")

  @functools.partial(
      pl.kernel,
      out_type=jax.ShapeDtypeStruct((n_rows, HIDDEN), jnp.float32),
      mesh=mesh,
      compiler_params=pltpu.CompilerParams(use_tc_tiling_on_sc=False),
      scratch_types=[
          pltpu.VMEM((rows_per_w,), jnp.int32),
          pltpu.VMEM((CHUNK, HIDDEN), jnp.float32),
          pltpu.VMEM((CHUNK, HIDDEN), jnp.float32),
          pltpu.SemaphoreType.DMA,
          pltpu.SemaphoreType.DMA,
          pltpu.SemaphoreType.DMA,
          pltpu.SemaphoreType.DMA,
      ],
  )
  def emb_kernel(table_hbm, idx_hbm, out_hbm, idx_v, buf0, buf1,
                 g_sem0, g_sem1, o_sem0, o_sem1):
    bufs = (buf0, buf1)
    g_sems = (g_sem0, g_sem1)
    o_sems = (o_sem0, o_sem1)

    wid = lax.axis_index("s") * NCORES + lax.axis_index("c")
    base = wid * rows_per_w

    pltpu.sync_copy(idx_hbm.at[pl.ds(base, rows_per_w)], idx_v)

    def gather(g, b):
      return pltpu.make_async_copy(
          table_hbm.at[idx_v.at[pl.ds(g * CHUNK, CHUNK)]], bufs[b], g_sems[b])

    def put(g, b):
      return pltpu.make_async_copy(
          bufs[b], out_hbm.at[pl.ds(base + g * CHUNK, CHUNK)], o_sems[b])

    gather(0, 0).start()
    for g in range(nchunks):
      b = g % 2
      if g + 1 < nchunks:
        if g >= 1:
          put(g - 1, 1 - b).wait()
        gather(g + 1, 1 - b).start()
      gather(g, b).wait()

      buf = bufs[b]

      @plsc.parallel_loop(0, CHUNK * vpr, 1, unroll=8)
      def scale_body(i):
        r = i // vpr
        c = (i % vpr) * LANES
        buf[r, pl.ds(c, LANES)] = buf[r, pl.ds(c, LANES)] * SCALE

      put(g, b).start()
    if nchunks >= 2:
      put(nchunks - 2, nchunks % 2).wait()
    put(nchunks - 1, (nchunks - 1) % 2).wait()

  return emb_kernel


@jax.jit
def kernel(x, table):
  n_rows = x.shape[0] * x.shape[1]
  flat_idx = x.reshape(n_rows)
  out = _make_kernel(n_rows)(table, flat_idx)
  return out.reshape(x.shape[0], x.shape[1], HIDDEN)


# speedup vs baseline: 1.4296x; 1.4296x over previous
import functools
import math

import jax
import jax.numpy as jnp
from jax import lax
from jax.experimental import pallas as pl
from jax.experimental.pallas import tpu as pltpu
from jax.experimental.pallas import tpu_sc as plsc

HIDDEN = 64
HIST = 20
SCALE = math.sqrt(HIDDEN)
LANES = 16
NCORES = 2
NSUB = 16
NW = NCORES * NSUB

CB = 16


def _make_kernel(batch: int):
  assert batch % (NW * CB) == 0
  items_per_w = batch // NW
  nchunks = items_per_w // CB
  rows_per_chunk = CB * HIST
  vpr = HIDDEN // LANES

  mesh = plsc.VectorSubcoreMesh(core_axis_name="c", subcore_axis_name="s")

  @functools.partial(
      pl.kernel,
      out_type=jax.ShapeDtypeStruct((batch, HIST, HIDDEN), jnp.float32),
      mesh=mesh,
      scratch_types=[
          pltpu.VMEM((CB * HIST,), jnp.int32),
          pltpu.VMEM((CB * HIST,), jnp.int32),
          pltpu.VMEM((CB, HIST, HIDDEN), jnp.float32),
          pltpu.VMEM((CB, HIST, HIDDEN), jnp.float32),
          pltpu.SemaphoreType.DMA,
          pltpu.SemaphoreType.DMA,
          pltpu.SemaphoreType.DMA,
          pltpu.SemaphoreType.DMA,
          pltpu.SemaphoreType.DMA,
          pltpu.SemaphoreType.DMA,
      ],
  )
  def emb_kernel(x_hbm, table_hbm, out_hbm, idx0, idx1, buf0, buf1,
                 gs0, gs1, os0, os1, is0, is1):
    idxs = (idx0, idx1)
    bufs = (buf0, buf1)
    g_sems = (gs0, gs1)
    o_sems = (os0, os1)
    i_sems = (is0, is1)

    wid = lax.axis_index("s") * NCORES + lax.axis_index("c")
    base = wid * items_per_w

    def idx_load(g, b):
      return pltpu.make_async_copy(
          x_hbm.at[pl.ds((base + g * CB) * HIST, CB * HIST)], idxs[b],
          i_sems[b])

    def put(g, b):
      return pltpu.make_async_copy(
          bufs[b], out_hbm.at[pl.ds(base + g * CB, CB)], o_sems[b])

    def issue_gathers(b):
      buf = bufs[b]
      sem = g_sems[b]
      idx_v = idxs[b]

      def body(vi, carry):
        v = idx_v[pl.ds(vi * LANES, LANES)]
        for l in range(LANES):
          r = vi * LANES + l
          bi = r // HIST
          h = r - bi * HIST
          i = v[l]
          pltpu.make_async_copy(
              table_hbm.at[pl.ds(i, 1)], buf.at[bi, pl.ds(h, 1)], sem).start()
        return carry

      lax.fori_loop(0, rows_per_chunk // LANES, body, 0)

    def drain_gathers(g, b):
      pltpu.make_async_copy(
          out_hbm.at[pl.ds(base + g * CB, CB)], bufs[b], g_sems[b]).wait()

    def scale(b):
      buf = bufs[b]

      @plsc.parallel_loop(0, rows_per_chunk * vpr, 1, unroll=8)
      def scale_body(i):
        r = i // vpr
        c = (i - r * vpr) * LANES
        bi = r // HIST
        h = r - bi * HIST
        buf[bi, h, pl.ds(c, LANES)] = buf[bi, h, pl.ds(c, LANES)] * SCALE

    idx_load(0, 0).start()
    idx_load(0, 0).wait()
    issue_gathers(0)
    if nchunks > 1:
      idx_load(1, 1).start()

    for g in range(nchunks):
      b = g % 2
      if g + 1 < nchunks:
        idx_load(g + 1, (g + 1) % 2).wait()
        if g >= 1:
          put(g - 1, 1 - b).wait()
        issue_gathers(1 - b)
        if g + 2 < nchunks:
          idx_load(g + 2, g % 2).start()
      drain_gathers(g, b)
      scale(b)
      put(g, b).start()

    if nchunks >= 2:
      put(nchunks - 2, nchunks % 2).wait()
    put(nchunks - 1, (nchunks - 1) % 2).wait()

  return emb_kernel


@jax.jit
def kernel(x, table):
  return _make_kernel(x.shape[0])(x.reshape(-1), table)
